# baseline (speedup 1.0000x reference)
import jax
import jax.numpy as jnp
from jax import lax
from jax.experimental import pallas as pl
from jax.experimental.pallas import tpu as pltpu
from jax.experimental.pallas import tpu_sc as plsc

_NUM_CORES = 2
_NUM_SUBCORES = 16
_NUM_WORKERS = _NUM_CORES * _NUM_SUBCORES
_CHUNK = 400


def kernel(indices, table):
    B, S = indices.shape
    V, D = table.shape
    N = B * S
    per_w = N // _NUM_WORKERS
    flat_idx = indices.reshape(N).astype(jnp.int32)
    pair_idx = flat_idx >> 1
    table2 = table.reshape(V // 2, 2 * D)
    mesh = plsc.VectorSubcoreMesh(core_axis_name="c", subcore_axis_name="s")

    @pl.kernel(
        out_type=jax.ShapeDtypeStruct((N, 2 * D), table.dtype),
        mesh=mesh,
        scratch_types=[
            pltpu.VMEM((_CHUNK,), jnp.int32),
            pltpu.VMEM((_CHUNK, 2 * D), jnp.float32),
            pltpu.SemaphoreType.DMA,
        ],
    )
    def gather_kernel(table_hbm, idx_hbm, out_hbm, idx_v, rows_v, sem):
        wid = lax.axis_index("s") * _NUM_CORES + lax.axis_index("c")
        base = wid * per_w

        @pl.loop(0, per_w, step=_CHUNK)
        def _(c0):
            pltpu.sync_copy(idx_hbm.at[pl.ds(base + c0, _CHUNK)], idx_v)
            pltpu.async_copy(table_hbm.at[idx_v], rows_v, sem).wait()
            pltpu.sync_copy(rows_v, out_hbm.at[pl.ds(base + c0, _CHUNK)])

    pairs = gather_kernel(table2, pair_idx)
    halves = pairs.reshape(N, 2, D)
    out = jnp.where((flat_idx & 1)[:, None] == 1, halves[:, 1, :], halves[:, 0, :])
    return out.reshape(B, S, D)


# speedup vs baseline: 1.6098x; 1.6098x over previous
import jax
import jax.numpy as jnp
from jax import lax
from jax.experimental import pallas as pl
from jax.experimental.pallas import tpu as pltpu
from jax.experimental.pallas import tpu_sc as plsc

_NUM_CORES = 2
_NUM_SUBCORES = 16
_NUM_WORKERS = _NUM_CORES * _NUM_SUBCORES
_CHUNK = 256
_LANES = 16


def kernel(indices, table):
    B, S = indices.shape
    V, D = table.shape
    N = B * S
    per_w = N // _NUM_WORKERS
    flat_idx = indices.reshape(N).astype(jnp.int32)
    table2 = table.reshape(V // 2, 2 * D)
    mesh = plsc.VectorSubcoreMesh(core_axis_name="c", subcore_axis_name="s")

    @pl.kernel(
        out_type=jax.ShapeDtypeStruct((N, D), table.dtype),
        mesh=mesh,
        scratch_types=[
            pltpu.VMEM((_CHUNK,), jnp.int32),
            pltpu.VMEM((_CHUNK,), jnp.int32),
            pltpu.VMEM((_CHUNK, 2 * D), jnp.float32),
            pltpu.VMEM((_CHUNK, D), jnp.float32),
            pltpu.SemaphoreType.DMA,
        ],
    )
    def gather_kernel(table_hbm, idx_hbm, out_hbm, idx_v, pair_v, pairs_v, o_v, sem):
        wid = lax.axis_index("s") * _NUM_CORES + lax.axis_index("c")
        base = wid * per_w

        @pl.loop(0, per_w, step=_CHUNK)
        def _(c0):
            pltpu.sync_copy(idx_hbm.at[pl.ds(base + c0, _CHUNK)], idx_v)

            @pl.loop(0, _CHUNK, step=_LANES)
            def _(k):
                sl = pl.ds(k, _LANES)
                pair_v[sl] = idx_v[sl] >> 1

            pltpu.async_copy(table_hbm.at[pair_v], pairs_v, sem).wait()

            @pl.loop(0, _CHUNK, step=_LANES)
            def _(k):
                offs = (idx_v[pl.ds(k, _LANES)] & 1) * D
                for j in range(_LANES):
                    off = offs[j]
                    for c in range(D // _LANES):
                        o_v[k + j, pl.ds(c * _LANES, _LANES)] = pairs_v[
                            k + j, pl.ds(off + c * _LANES, _LANES)
                        ]

            pltpu.sync_copy(o_v, out_hbm.at[pl.ds(base + c0, _CHUNK)])

    out = gather_kernel(table2, flat_idx)
    return out.reshape(B, S, D)


# speedup vs baseline: 2.5788x; 1.6020x over previous
import jax
import jax.numpy as jnp
from jax import lax
from jax.experimental import pallas as pl
from jax.experimental.pallas import tpu as pltpu
from jax.experimental.pallas import tpu_sc as plsc

_NUM_CORES = 2
_NUM_SUBCORES = 16
_NUM_WORKERS = _NUM_CORES * _NUM_SUBCORES
_CHUNK = 400
_LANES = 16


def kernel(indices, table):
    B, S = indices.shape
    V, D = table.shape
    N = B * S
    per_w = N // _NUM_WORKERS
    flat_idx = indices.reshape(N).astype(jnp.int32)
    mesh = plsc.VectorSubcoreMesh(core_axis_name="c", subcore_axis_name="s")

    @pl.kernel(
        out_type=jax.ShapeDtypeStruct((N, D), table.dtype),
        mesh=mesh,
        scratch_types=[
            pltpu.VMEM((_CHUNK,), jnp.int32),
            pltpu.VMEM((_CHUNK, D), jnp.float32),
            pltpu.SemaphoreType.DMA,
        ],
    )
    def gather_kernel(table_hbm, idx_hbm, out_hbm, idx_v, rows_v, sem):
        wid = lax.axis_index("s") * _NUM_CORES + lax.axis_index("c")
        base = wid * per_w

        @pl.loop(0, per_w, step=_CHUNK)
        def _(c0):
            pltpu.sync_copy(idx_hbm.at[pl.ds(base + c0, _CHUNK)], idx_v)

            @pl.loop(0, _CHUNK, step=_LANES)
            def _(k):
                vec = idx_v[pl.ds(k, _LANES)]
                for j in range(_LANES):
                    pltpu.make_async_copy(
                        table_hbm.at[pl.ds(vec[j], 1)],
                        rows_v.at[pl.ds(k + j, 1)],
                        sem,
                    ).start()

            pltpu.make_async_copy(
                table_hbm.at[pl.ds(0, _CHUNK)], rows_v, sem
            ).wait()

            pltpu.sync_copy(rows_v, out_hbm.at[pl.ds(base + c0, _CHUNK)])

    out = gather_kernel(table, flat_idx)
    return out.reshape(B, S, D)
